# baseline (speedup 1.0000x reference)
import functools

import jax
import jax.numpy as jnp
from jax.experimental import pallas as pl
from jax.experimental.pallas import tpu as pltpu

B, T = 16, 32
VOCAB, EMBED, HIDDEN = 10000, 128, 512
SLOTS, DIM, HEADS, TOPK = 2048, 64, 4, 8
SEFF = 256
IN_DIM = EMBED + DIM

_NT = (((1,), (1,)), ((), ()))
_NN = (((1,), (0,)), ((), ()))


def _recurrence_kernel(seq_ref, betas_ref, emb_ref, wxtop_ref, wxrd_ref,
                       whzr_ref, whn_ref, bzr_ref, bn_ref, wkey_ref, bkey_ref,
                       wert_ref, berc_ref, wwvt_ref, bwvc_ref, wagr_ref,
                       hseq_ref, memt_ref, eproj_ref, xemb_ref):
    f32 = jnp.float32
    beta_r = jnp.clip(jax.nn.softplus(betas_ref[0, 0]), 1.0, 20.0)
    beta_w = jnp.clip(jax.nn.softplus(betas_ref[0, 1]), 1.0, 20.0)
    bag = betas_ref[0, 2]

    memt_ref[...] = jnp.zeros((B, DIM, SEFF), f32)

    def _gather(i, _):
        t = i // B
        b = i - t * B
        idx = seq_ref[t, b]
        xemb_ref[pl.ds(i, 1), :] = emb_ref[pl.ds(idx, 1), :]
        return _

    jax.lax.fori_loop(0, T * B, _gather, 0, unroll=8)

    eproj_ref[...] = jax.lax.dot_general(
        xemb_ref[...], wxtop_ref[...], _NN, preferred_element_type=f32)

    NR = B * 2 * HEADS
    hmod = jax.lax.broadcasted_iota(jnp.int32, (NR, 1), 0) % (2 * HEADS)
    beta_rows = jnp.where(hmod < HEADS, beta_r, beta_w)
    rr = jax.lax.broadcasted_iota(jnp.int32, (NR, NR), 0)
    cc = jax.lax.broadcasted_iota(jnp.int32, (NR, NR), 1)
    perm = (cc == (rr % (2 * HEADS)) * B + rr // (2 * HEADS)).astype(f32)
    rq = jax.lax.broadcasted_iota(jnp.int32, (B, NR), 0)
    cq = jax.lax.broadcasted_iota(jnp.int32, (B, NR), 1)
    hq = cq - rq * (2 * HEADS)
    qsel = jnp.where((hq >= HEADS) & (hq < 2 * HEADS), 0.25, 0.0)

    def step(t, carry):
        h, read_vec = carry

        zrn_x = (eproj_ref[pl.ds(t * B, B), :]
                 + jax.lax.dot_general(read_vec, wxrd_ref[...], _NN,
                                       preferred_element_type=f32))
        zr = zrn_x[:, :2 * HIDDEN] + jax.lax.dot_general(
            h, whzr_ref[...], _NN, preferred_element_type=f32) + bzr_ref[...]
        z = jax.nn.sigmoid(zr[:, :HIDDEN])
        r = jax.nn.sigmoid(zr[:, HIDDEN:])
        n = jnp.tanh(zrn_x[:, 2 * HIDDEN:]
                     + jax.lax.dot_general(r * h, whn_ref[...], _NN,
                                           preferred_element_type=f32)
                     + bn_ref[...])
        h_new = (1.0 - z) * n + z * h

        keys = jax.lax.dot_general(h_new, wkey_ref[...], _NN,
                                   preferred_element_type=f32) + bkey_ref[...]

        erT = jax.nn.sigmoid(
            jax.lax.dot_general(wert_ref[...], h_new, _NT,
                                preferred_element_type=f32) + berc_ref[...])
        wvT = jax.lax.dot_general(wwvt_ref[...], h_new, _NT,
                                  preferred_element_type=f32) + bwvc_ref[...]
        agT = jax.nn.sigmoid(
            jax.lax.dot_general(wagr_ref[...], h_new, _NT,
                                preferred_element_type=f32) + bag)
        addT = wvT * agT

        kstack = jnp.concatenate(
            [keys[:, hh * DIM:(hh + 1) * DIM]
             for hh in range(2 * HEADS)], axis=0)
        kall = jax.lax.dot_general(perm, kstack, _NN,
                                   preferred_element_type=f32)
        kall = kall * jax.lax.rsqrt(
            jnp.sum(kall * kall, axis=1, keepdims=True) + 1e-6) * beta_rows

        sims = []
        for b in range(B):
            memb = memt_ref[b]
            invn = jax.lax.rsqrt(
                jnp.sum(memb * memb, axis=0, keepdims=True) + 1e-6)
            kb = kall[b * 2 * HEADS:(b + 1) * 2 * HEADS, :]
            sims.append(jax.lax.dot_general(
                kb, memb, _NN, preferred_element_type=f32) * invn)
        sim = jnp.concatenate(sims, axis=0)

        iota_s = jax.lax.broadcasted_iota(jnp.int32, (B * 2 * HEADS, SEFF), 1)
        wacc = jnp.zeros((B * 2 * HEADS, SEFF), f32)
        zsum = jnp.zeros((B * 2 * HEADS, 1), f32)
        v0 = None
        for k in range(TOPK):
            m = jnp.max(sim, axis=1, keepdims=True)
            cand = jnp.where(sim == m, iota_s, SEFF)
            amin = jnp.min(cand, axis=1, keepdims=True)
            if v0 is None:
                v0 = m
                e = jnp.ones((B * 2 * HEADS, 1), f32)
            else:
                e = jnp.exp(m - v0)
            hit = iota_s == amin
            wacc = jnp.where(hit, e, wacc)
            zsum = zsum + e
            if k < TOPK - 1:
                sim = jnp.where(hit, -1e30, sim)
        w = wacc * (1.0 / zsum)

        ww_all = jax.lax.dot_general(qsel, w, _NN,
                                     preferred_element_type=f32)
        reads = []
        mems = []
        for b in range(B):
            memb = memt_ref[b]
            wr = w[b * 2 * HEADS:b * 2 * HEADS + HEADS, :]
            rb = jax.lax.dot_general(wr, memb, _NT,
                                     preferred_element_type=f32)
            reads.append(jnp.mean(rb, axis=0, keepdims=True))
            wwb = ww_all[b:b + 1, :]
            erb = erT[:, b:b + 1]
            addb = addT[:, b:b + 1]
            mems.append((memb - wwb * (erb * memb - addb))
                        .reshape(1, DIM, SEFF))
        memt_ref[...] = jnp.concatenate(mems, axis=0)
        read_new = jnp.concatenate(reads, axis=0)

        hseq_ref[pl.ds(t, 1)] = h_new.reshape(1, B, HIDDEN)
        return h_new, read_new

    h0 = jnp.zeros((B, HIDDEN), f32)
    r0 = jnp.zeros((B, DIM), f32)
    jax.lax.fori_loop(0, T, step, (h0, r0), unroll=2)


def _logits_kernel(h_ref, w_ref, b_ref, o_ref):
    o_ref[...] = jax.lax.dot_general(
        h_ref[...], w_ref[...], _NN,
        preferred_element_type=jnp.float32) + b_ref[...]


@functools.partial(jax.jit, static_argnames=())
def kernel(input_seq, embedding, Wxz, Whz, bz, Wxr, Whr, br, Wxn, Whn, bn,
           W_out, b_out, W_rk, b_rk, W_wk, b_wk, W_wv, b_wv, W_er, b_er,
           W_ag, b_ag, beta_read, beta_write):
    f32 = jnp.float32
    seq = input_seq.astype(jnp.int32)

    wx = jnp.concatenate([Wxz, Wxr, Wxn], axis=1)
    wxtop = wx[:EMBED, :]
    wxrd = wx[EMBED:, :]
    whzr = jnp.concatenate([Whz, Whr], axis=1)
    bzr = jnp.concatenate([bz, br]).reshape(1, 2 * HIDDEN)
    bn2 = bn.reshape(1, HIDDEN)
    wkey = jnp.concatenate([W_rk, W_wk], axis=1)
    bkey = jnp.concatenate([b_rk, b_wk]).reshape(1, 2 * HEADS * DIM)
    wert = W_er.T
    berc = b_er.reshape(DIM, 1)
    wwvt = W_wv.T
    bwvc = b_wv.reshape(DIM, 1)
    wagr = W_ag.T
    betas = jnp.stack(
        [beta_read, beta_write, b_ag[0]]).reshape(1, 3).astype(f32)

    smem = pl.BlockSpec(memory_space=pltpu.SMEM)
    vmem = pl.BlockSpec(memory_space=pltpu.VMEM)

    hseq = pl.pallas_call(
        _recurrence_kernel,
        out_shape=jax.ShapeDtypeStruct((T, B, HIDDEN), f32),
        in_specs=[smem, smem] + [vmem] * 14,
        out_specs=vmem,
        scratch_shapes=[
            pltpu.VMEM((B, DIM, SEFF), f32),
            pltpu.VMEM((T * B, 3 * HIDDEN), f32),
            pltpu.VMEM((T * B, EMBED), f32),
        ],
        compiler_params=pltpu.CompilerParams(
            vmem_limit_bytes=60 * 1024 * 1024),
    )(seq.T, betas, embedding, wxtop, wxrd, whzr, Whn, bzr, bn2, wkey, bkey,
      wert, berc, wwvt, bwvc, wagr)

    hflat = hseq.transpose(1, 0, 2).reshape(B * T, HIDDEN)

    vt = 2048
    ntiles = (VOCAB + vt - 1) // vt
    logits = pl.pallas_call(
        _logits_kernel,
        grid=(ntiles,),
        out_shape=jax.ShapeDtypeStruct((B * T, VOCAB), f32),
        in_specs=[
            pl.BlockSpec((B * T, HIDDEN), lambda j: (0, 0)),
            pl.BlockSpec((HIDDEN, vt), lambda j: (0, j)),
            pl.BlockSpec((1, vt), lambda j: (0, j)),
        ],
        out_specs=pl.BlockSpec((B * T, vt), lambda j: (0, j)),
        compiler_params=pltpu.CompilerParams(
            vmem_limit_bytes=60 * 1024 * 1024),
    )(hflat, W_out, b_out.reshape(1, VOCAB))

    return logits.reshape(B, T, VOCAB)


# speedup vs baseline: 1.0031x; 1.0031x over previous
import functools

import jax
import jax.numpy as jnp
from jax.experimental import pallas as pl
from jax.experimental.pallas import tpu as pltpu

B, T = 16, 32
VOCAB, EMBED, HIDDEN = 10000, 128, 512
SLOTS, DIM, HEADS, TOPK = 2048, 64, 4, 8
SEFF = 256
IN_DIM = EMBED + DIM

_NT = (((1,), (1,)), ((), ()))
_NN = (((1,), (0,)), ((), ()))


def _recurrence_kernel(seq_ref, betas_ref, emb_ref, wxtop_ref, wxrd_ref,
                       whzr_ref, whn_ref, bzr_ref, bn_ref, wkey_ref, bkey_ref,
                       wert_ref, berc_ref, wwvt_ref, bwvc_ref, wagr_ref,
                       hseq_ref, memt_ref, eproj_ref, xemb_ref):
    f32 = jnp.float32
    beta_r = jnp.clip(jax.nn.softplus(betas_ref[0, 0]), 1.0, 20.0)
    beta_w = jnp.clip(jax.nn.softplus(betas_ref[0, 1]), 1.0, 20.0)
    bag = betas_ref[0, 2]

    memt_ref[...] = jnp.zeros((B, DIM, SEFF), f32)

    def _gather(i, _):
        t = i // B
        b = i - t * B
        idx = seq_ref[t, b]
        xemb_ref[pl.ds(i, 1), :] = emb_ref[pl.ds(idx, 1), :]
        return _

    jax.lax.fori_loop(0, T * B, _gather, 0, unroll=8)

    eproj_ref[...] = jax.lax.dot_general(
        xemb_ref[...], wxtop_ref[...], _NN, preferred_element_type=f32)

    NR = B * 2 * HEADS
    hmod = jax.lax.broadcasted_iota(jnp.int32, (NR, 1), 0) % (2 * HEADS)
    beta_rows = jnp.where(hmod < HEADS, beta_r, beta_w)
    rr = jax.lax.broadcasted_iota(jnp.int32, (NR, NR), 0)
    cc = jax.lax.broadcasted_iota(jnp.int32, (NR, NR), 1)
    perm = (cc == (rr % (2 * HEADS)) * B + rr // (2 * HEADS)).astype(f32)

    def step(t, carry):
        h, read_vec = carry

        zrn_x = (eproj_ref[pl.ds(t * B, B), :]
                 + jax.lax.dot_general(read_vec, wxrd_ref[...], _NN,
                                       preferred_element_type=f32))
        zr = zrn_x[:, :2 * HIDDEN] + jax.lax.dot_general(
            h, whzr_ref[...], _NN, preferred_element_type=f32) + bzr_ref[...]
        z = jax.nn.sigmoid(zr[:, :HIDDEN])
        r = jax.nn.sigmoid(zr[:, HIDDEN:])
        n = jnp.tanh(zrn_x[:, 2 * HIDDEN:]
                     + jax.lax.dot_general(r * h, whn_ref[...], _NN,
                                           preferred_element_type=f32)
                     + bn_ref[...])
        h_new = (1.0 - z) * n + z * h

        keys = jax.lax.dot_general(h_new, wkey_ref[...], _NN,
                                   preferred_element_type=f32) + bkey_ref[...]

        erT = jax.nn.sigmoid(
            jax.lax.dot_general(wert_ref[...], h_new, _NT,
                                preferred_element_type=f32) + berc_ref[...])
        wvT = jax.lax.dot_general(wwvt_ref[...], h_new, _NT,
                                  preferred_element_type=f32) + bwvc_ref[...]
        agT = jax.nn.sigmoid(
            jax.lax.dot_general(wagr_ref[...], h_new, _NT,
                                preferred_element_type=f32) + bag)
        addT = wvT * agT

        kstack = jnp.concatenate(
            [keys[:, hh * DIM:(hh + 1) * DIM]
             for hh in range(2 * HEADS)], axis=0)
        kall = jax.lax.dot_general(perm, kstack, _NN,
                                   preferred_element_type=f32)
        kall = kall * jax.lax.rsqrt(
            jnp.sum(kall * kall, axis=1, keepdims=True) + 1e-6) * beta_rows

        sims = []
        for b in range(B):
            memb = memt_ref[b]
            invn = jax.lax.rsqrt(
                jnp.sum(memb * memb, axis=0, keepdims=True) + 1e-6)
            kb = kall[b * 2 * HEADS:(b + 1) * 2 * HEADS, :]
            sims.append(jax.lax.dot_general(
                kb, memb, _NN, preferred_element_type=f32) * invn)
        sim = jnp.concatenate(sims, axis=0)

        iota_s = jax.lax.broadcasted_iota(jnp.int32, (B * 2 * HEADS, SEFF), 1)
        wacc = jnp.zeros((B * 2 * HEADS, SEFF), f32)
        zsum = jnp.zeros((B * 2 * HEADS, 1), f32)
        v0 = None
        for k in range(TOPK):
            m = jnp.max(sim, axis=1, keepdims=True)
            cand = jnp.where(sim == m, iota_s, SEFF)
            amin = jnp.min(cand, axis=1, keepdims=True)
            if v0 is None:
                v0 = m
                e = jnp.ones((B * 2 * HEADS, 1), f32)
            else:
                e = jnp.exp(m - v0)
            hit = iota_s == amin
            wacc = jnp.where(hit, e, wacc)
            zsum = zsum + e
            if k < TOPK - 1:
                sim = jnp.where(hit, -1e30, sim)
        w = wacc * (1.0 / zsum)

        reads = []
        for b in range(B):
            memb = memt_ref[b]
            wr = w[b * 2 * HEADS:b * 2 * HEADS + HEADS, :]
            rb = jax.lax.dot_general(wr, memb, _NT,
                                     preferred_element_type=f32)
            reads.append(jnp.mean(rb, axis=0, keepdims=True))
            wwb = jnp.mean(w[b * 2 * HEADS + HEADS:(b + 1) * 2 * HEADS, :],
                           axis=0, keepdims=True)
            erb = erT[:, b:b + 1]
            addb = addT[:, b:b + 1]
            memt_ref[b] = memb - wwb * (erb * memb - addb)
        read_new = jnp.concatenate(reads, axis=0)

        hseq_ref[pl.ds(t, 1)] = h_new.reshape(1, B, HIDDEN)
        return h_new, read_new

    h0 = jnp.zeros((B, HIDDEN), f32)
    r0 = jnp.zeros((B, DIM), f32)
    jax.lax.fori_loop(0, T, step, (h0, r0), unroll=2)


def _logits_kernel(h_ref, w_ref, b_ref, o_ref):
    o_ref[...] = jax.lax.dot_general(
        h_ref[...], w_ref[...], _NN,
        preferred_element_type=jnp.float32) + b_ref[...]


@functools.partial(jax.jit, static_argnames=())
def kernel(input_seq, embedding, Wxz, Whz, bz, Wxr, Whr, br, Wxn, Whn, bn,
           W_out, b_out, W_rk, b_rk, W_wk, b_wk, W_wv, b_wv, W_er, b_er,
           W_ag, b_ag, beta_read, beta_write):
    f32 = jnp.float32
    seq = input_seq.astype(jnp.int32)

    wx = jnp.concatenate([Wxz, Wxr, Wxn], axis=1)
    wxtop = wx[:EMBED, :]
    wxrd = wx[EMBED:, :]
    whzr = jnp.concatenate([Whz, Whr], axis=1)
    bzr = jnp.concatenate([bz, br]).reshape(1, 2 * HIDDEN)
    bn2 = bn.reshape(1, HIDDEN)
    wkey = jnp.concatenate([W_rk, W_wk], axis=1)
    bkey = jnp.concatenate([b_rk, b_wk]).reshape(1, 2 * HEADS * DIM)
    wert = W_er.T
    berc = b_er.reshape(DIM, 1)
    wwvt = W_wv.T
    bwvc = b_wv.reshape(DIM, 1)
    wagr = W_ag.T
    betas = jnp.stack(
        [beta_read, beta_write, b_ag[0]]).reshape(1, 3).astype(f32)

    smem = pl.BlockSpec(memory_space=pltpu.SMEM)
    vmem = pl.BlockSpec(memory_space=pltpu.VMEM)

    hseq = pl.pallas_call(
        _recurrence_kernel,
        out_shape=jax.ShapeDtypeStruct((T, B, HIDDEN), f32),
        in_specs=[smem, smem] + [vmem] * 14,
        out_specs=vmem,
        scratch_shapes=[
            pltpu.VMEM((B, DIM, SEFF), f32),
            pltpu.VMEM((T * B, 3 * HIDDEN), f32),
            pltpu.VMEM((T * B, EMBED), f32),
        ],
        compiler_params=pltpu.CompilerParams(
            vmem_limit_bytes=60 * 1024 * 1024),
    )(seq.T, betas, embedding, wxtop, wxrd, whzr, Whn, bzr, bn2, wkey, bkey,
      wert, berc, wwvt, bwvc, wagr)

    hflat = hseq.transpose(1, 0, 2).reshape(B * T, HIDDEN)

    vt = 2048
    ntiles = (VOCAB + vt - 1) // vt
    logits = pl.pallas_call(
        _logits_kernel,
        grid=(ntiles,),
        out_shape=jax.ShapeDtypeStruct((B * T, VOCAB), f32),
        in_specs=[
            pl.BlockSpec((B * T, HIDDEN), lambda j: (0, 0)),
            pl.BlockSpec((HIDDEN, vt), lambda j: (0, j)),
            pl.BlockSpec((1, vt), lambda j: (0, j)),
        ],
        out_specs=pl.BlockSpec((B * T, vt), lambda j: (0, j)),
        compiler_params=pltpu.CompilerParams(
            vmem_limit_bytes=60 * 1024 * 1024),
    )(hflat, W_out, b_out.reshape(1, VOCAB))

    return logits.reshape(B, T, VOCAB)
